# baseline (speedup 1.0000x reference)
import functools

import jax
import jax.numpy as jnp
from jax import lax
from jax.experimental import pallas as pl
from jax.experimental.pallas import tpu as pltpu
from jax.experimental.pallas import tpu_sc as plsc

_BINS = 8
_MIN_W = 0.001
_MIN_H = 0.001
_MIN_D = 0.001
_TAIL = 3.0
_N = 2097152
_P = 3 * _BINS + 1

_NC, _NS, _L = 2, 16, 16
_NW = _NC * _NS
_ROWS_PER_W = _N // _NW
_CHUNK = 2048
_NCHUNK = _ROWS_PER_W // _CHUNK
_GROUPS = _CHUNK // _L

_LN2 = 0.6931471805599453


def _log_f32(x):
    bits = lax.bitcast_convert_type(x, jnp.int32)
    e = (bits >> 23) & 0xFF
    m_bits = (bits & 0x007FFFFF) | 0x3F800000
    m = lax.bitcast_convert_type(m_bits, jnp.float32)
    big = m > jnp.float32(1.4142135)
    m = jnp.where(big, m * jnp.float32(0.5), m)
    e = jnp.where(big, e - 126, e - 127)
    ef = e.astype(jnp.float32)
    s = (m - 1.0) / (m + 1.0)
    s2 = s * s
    p = jnp.float32(2.0) * s * (
        1.0 + s2 * (jnp.float32(1.0 / 3.0) + s2 * (jnp.float32(0.2)
            + s2 * (jnp.float32(1.0 / 7.0) + s2 * jnp.float32(1.0 / 9.0)))))
    return ef * jnp.float32(_LN2) + p


def _softplus(x):
    return jnp.maximum(x, 0.0) + _log_f32(1.0 + jnp.exp(-jnp.abs(x)))


def _rqs_group(theta_v, y_v, tu_v, ld_v, g):
    row = g * _L + lax.broadcasted_iota(jnp.int32, (_L,), 0)
    flat = row * _P

    uw = [plsc.load_gather(theta_v, [flat + j]) for j in range(_BINS)]
    uh = [plsc.load_gather(theta_v, [flat + (_BINS + j)]) for j in range(_BINS)]
    ud = [plsc.load_gather(theta_v, [flat + (2 * _BINS + j)])
          for j in range(1, _BINS)]
    y = y_v[pl.ds(g * _L, _L)]

    def norm_bins(u, min_b):
        m = u[0]
        for j in range(1, _BINS):
            m = jnp.maximum(m, u[j])
        e = [jnp.exp(u[j] - m) for j in range(_BINS)]
        s = e[0]
        for j in range(1, _BINS):
            s = s + e[j]
        rinv = 1.0 / s
        c = jnp.float32(1.0 - min_b * _BINS)
        return [jnp.float32(min_b) + c * e[j] * rinv for j in range(_BINS)]

    w = norm_bins(uw, _MIN_W)
    h = norm_bins(uh, _MIN_H)

    def knots(v):
        cw = [jnp.full((_L,), -_TAIL, jnp.float32)]
        s = v[0]
        for j in range(1, _BINS):
            cw.append(jnp.float32(2.0 * _TAIL) * s - _TAIL)
            s = s + v[j]
        cw.append(jnp.full((_L,), _TAIL, jnp.float32))
        return cw

    cw = knots(w)
    ch = knots(h)
    W = [cw[k + 1] - cw[k] for k in range(_BINS)]
    H = [ch[k + 1] - ch[k] for k in range(_BINS)]
    one = jnp.full((_L,), 1.0, jnp.float32)
    d = [one] + [jnp.float32(_MIN_D) + _softplus(u) for u in ud] + [one]

    yc = jnp.clip(y, -_TAIL, _TAIL)
    q_cw, q_W, q_ch, q_H, q_d, q_d1 = cw[0], W[0], ch[0], H[0], d[0], d[1]
    for k in range(1, _BINS):
        ge = yc >= cw[k]
        q_cw = jnp.where(ge, cw[k], q_cw)
        q_W = jnp.where(ge, W[k], q_W)
        q_ch = jnp.where(ge, ch[k], q_ch)
        q_H = jnp.where(ge, H[k], q_H)
        q_d = jnp.where(ge, d[k], q_d)
        q_d1 = jnp.where(ge, d[k + 1], q_d1)

    delta = q_H / q_W
    th = (yc - q_cw) / q_W
    th_om = th * (1.0 - th)
    th2 = th * th
    num = q_H * (delta * th2 + q_d * th_om)
    den = delta + (q_d + q_d1 - 2.0 * delta) * th_om
    out_in = q_ch + num / den
    omth = 1.0 - th
    dnum = delta * delta * (q_d1 * th2 + 2.0 * delta * th_om + q_d * omth * omth)
    ld_in = _log_f32(dnum / (den * den))

    inside = (y >= -_TAIL) & (y <= _TAIL)
    tu_v[pl.ds(g * _L, _L)] = jnp.where(inside, out_in, y)
    ld_v[pl.ds(g * _L, _L)] = jnp.where(inside, ld_in, jnp.float32(0.0))


def _sc_body(theta_hbm, y_hbm, tu_hbm, ld_hbm, theta_v, y_v, tu_v, ld_v):
    wid = lax.axis_index("s") * _NC + lax.axis_index("c")
    base = wid * _ROWS_PER_W

    def chunk_body(ci, carry):
        cbase = base + ci * _CHUNK
        pltpu.sync_copy(theta_hbm.at[pl.ds(cbase * _P, _CHUNK * _P)], theta_v)
        pltpu.sync_copy(y_hbm.at[pl.ds(cbase, _CHUNK)], y_v)

        def group_body(g, c):
            _rqs_group(theta_v, y_v, tu_v, ld_v, g)
            return c

        lax.fori_loop(0, _GROUPS, group_body, 0)
        pltpu.sync_copy(tu_v, tu_hbm.at[pl.ds(cbase, _CHUNK)])
        pltpu.sync_copy(ld_v, ld_hbm.at[pl.ds(cbase, _CHUNK)])
        return carry

    lax.fori_loop(0, _NCHUNK, chunk_body, 0)


_rqs_sc = functools.partial(
    pl.kernel,
    out_type=(
        jax.ShapeDtypeStruct((_N,), jnp.float32),
        jax.ShapeDtypeStruct((_N,), jnp.float32),
    ),
    mesh=plsc.VectorSubcoreMesh(core_axis_name="c", subcore_axis_name="s"),
    compiler_params=pltpu.CompilerParams(needs_layout_passes=False),
    scratch_types=[
        pltpu.VMEM((_CHUNK * _P,), jnp.float32),
        pltpu.VMEM((_CHUNK,), jnp.float32),
        pltpu.VMEM((_CHUNK,), jnp.float32),
        pltpu.VMEM((_CHUNK,), jnp.float32),
    ],
)(_sc_body)


_TCB = 8192
_TCS = 1024


def _softplus_tc(x):
    return jnp.maximum(x, 0.0) + jnp.log1p(jnp.exp(-jnp.abs(x)))


def _tc_body(theta_ref, y_ref, tu_ref, ld_ref):
    def sub_tile(i, carry):
        _tc_subtile(theta_ref, y_ref, tu_ref, ld_ref, i)
        return carry

    lax.fori_loop(0, _TCB // _TCS, sub_tile, 0, unroll=16)


def _tc_subtile(theta_ref, y_ref, tu_ref, ld_ref, i):
    t = theta_ref[pl.ds(i * _TCS, _TCS), :]
    f32 = jnp.float32
    tt = t.T
    uw = tt[0:8]
    uh = tt[8:16]
    udm = tt[17:24]
    y = y_ref[pl.ds(i * _TCS, _TCS)].reshape(1, _TCS)

    def norm_bins(u, scale, minb):
        e = jnp.exp(u)
        s = jnp.sum(e, axis=0, keepdims=True)
        rinv = f32((1.0 - minb * _BINS) * scale) / s
        return f32(minb * scale) + e * rinv

    w6 = norm_bins(uw, 2.0 * _TAIL, _MIN_W)
    h6 = norm_bins(uh, 2.0 * _TAIL, _MIN_H)

    i7 = lax.broadcasted_iota(jnp.int32, (7, _BINS), 0)
    j7 = lax.broadcasted_iota(jnp.int32, (7, _BINS), 1)
    ltri = (j7 <= i7).astype(f32)
    cs = lax.dot_general(ltri, w6, (((1,), (0,)), ((), ())),
                         preferred_element_type=f32,
                         precision=lax.Precision.HIGHEST)

    yc = jnp.clip(y, -_TAIL, _TAIL)
    z = yc + f32(_TAIL)
    ge = jnp.where(z >= cs, f32(1.0), f32(0.0))

    dm = f32(_MIN_D) + _softplus_tc(udm)
    one = jnp.ones((1, udm.shape[1]), f32)
    dmsh = jnp.concatenate([one, dm[0:6]], axis=0)
    dmsh1 = jnp.concatenate([dm[1:7], one], axis=0)

    def acc(diff):
        return jnp.sum(ge * diff, axis=0, keepdims=True)

    q_cw = acc(w6[0:7]) - f32(_TAIL)
    q_ch = acc(h6[0:7]) - f32(_TAIL)
    q_w = w6[0:1] + acc(w6[1:8] - w6[0:7])
    q_h = h6[0:1] + acc(h6[1:8] - h6[0:7])
    q_d = f32(1.0) + acc(dm - dmsh)
    q_d1 = dm[0:1] + acc(dmsh1 - dm)

    winv = f32(1.0) / q_w
    delta = q_h * winv
    th = (yc - q_cw) * winv
    th_om = th * (1.0 - th)
    th2 = th * th
    num = q_h * (delta * th2 + q_d * th_om)
    den = delta + (q_d + q_d1 - 2.0 * delta) * th_om
    deninv = f32(1.0) / den
    out_in = q_ch + num * deninv
    omth = 1.0 - th
    dnum = delta * delta * (q_d1 * th2 + 2.0 * delta * th_om + q_d * omth * omth)
    ld_in = jnp.log(dnum * deninv * deninv)

    inside = (y >= -_TAIL) & (y <= _TAIL)
    tu_ref[pl.ds(i * _TCS, _TCS)] = jnp.where(inside, out_in, y).reshape(_TCS)
    ld_ref[pl.ds(i * _TCS, _TCS)] = jnp.where(inside, ld_in, f32(0.0)).reshape(_TCS)


def _rqs_tc(theta, y):
    n = theta.shape[0]
    half = n // 2
    nb = half // _TCB

    def body(ta_ref, tb_ref, ya_ref, yb_ref, tua_ref, tub_ref, lda_ref, ldb_ref):
        def sub_tile(i, carry):
            _tc_subtile(ta_ref, ya_ref, tua_ref, lda_ref, i)
            _tc_subtile(tb_ref, yb_ref, tub_ref, ldb_ref, i)
            return carry

        lax.fori_loop(0, _TCB // _TCS, sub_tile, 0, unroll=8)

    tua, tub, lda, ldb = pl.pallas_call(
        body,
        grid=(nb,),
        in_specs=[
            pl.BlockSpec((_TCB, _P), lambda i: (i, 0)),
            pl.BlockSpec((_TCB, _P), lambda i: (i + nb, 0)),
            pl.BlockSpec((_TCB,), lambda i: (i,)),
            pl.BlockSpec((_TCB,), lambda i: (i + nb,)),
        ],
        out_specs=[
            pl.BlockSpec((_TCB,), lambda i: (i,)),
            pl.BlockSpec((_TCB,), lambda i: (i,)),
            pl.BlockSpec((_TCB,), lambda i: (i,)),
            pl.BlockSpec((_TCB,), lambda i: (i,)),
        ],
        out_shape=[
            jax.ShapeDtypeStruct((half,), jnp.float32),
            jax.ShapeDtypeStruct((half,), jnp.float32),
            jax.ShapeDtypeStruct((half,), jnp.float32),
            jax.ShapeDtypeStruct((half,), jnp.float32),
        ],
    )(theta, theta, y, y)
    tu = jnp.concatenate([tua, tub])
    ld = jnp.concatenate([lda, ldb])
    return tu, ld


def kernel(theta, y):
    tu, ld = _rqs_tc(theta, y)
    return tu.reshape(-1, 1), ld


# speedup vs baseline: 1.0218x; 1.0218x over previous
import functools

import jax
import jax.numpy as jnp
from jax import lax
from jax.experimental import pallas as pl
from jax.experimental.pallas import tpu as pltpu
from jax.experimental.pallas import tpu_sc as plsc

_BINS = 8
_MIN_W = 0.001
_MIN_H = 0.001
_MIN_D = 0.001
_TAIL = 3.0
_N = 2097152
_P = 3 * _BINS + 1

_NC, _NS, _L = 2, 16, 16
_NW = _NC * _NS
_ROWS_PER_W = _N // _NW
_CHUNK = 2048
_NCHUNK = _ROWS_PER_W // _CHUNK
_GROUPS = _CHUNK // _L

_LN2 = 0.6931471805599453


def _log_f32(x):
    bits = lax.bitcast_convert_type(x, jnp.int32)
    e = (bits >> 23) & 0xFF
    m_bits = (bits & 0x007FFFFF) | 0x3F800000
    m = lax.bitcast_convert_type(m_bits, jnp.float32)
    big = m > jnp.float32(1.4142135)
    m = jnp.where(big, m * jnp.float32(0.5), m)
    e = jnp.where(big, e - 126, e - 127)
    ef = e.astype(jnp.float32)
    s = (m - 1.0) / (m + 1.0)
    s2 = s * s
    p = jnp.float32(2.0) * s * (
        1.0 + s2 * (jnp.float32(1.0 / 3.0) + s2 * (jnp.float32(0.2)
            + s2 * (jnp.float32(1.0 / 7.0) + s2 * jnp.float32(1.0 / 9.0)))))
    return ef * jnp.float32(_LN2) + p


def _softplus(x):
    return jnp.maximum(x, 0.0) + _log_f32(1.0 + jnp.exp(-jnp.abs(x)))


def _rqs_group(theta_v, y_v, tu_v, ld_v, g):
    row = g * _L + lax.broadcasted_iota(jnp.int32, (_L,), 0)
    flat = row * _P

    uw = [plsc.load_gather(theta_v, [flat + j]) for j in range(_BINS)]
    uh = [plsc.load_gather(theta_v, [flat + (_BINS + j)]) for j in range(_BINS)]
    ud = [plsc.load_gather(theta_v, [flat + (2 * _BINS + j)])
          for j in range(1, _BINS)]
    y = y_v[pl.ds(g * _L, _L)]

    def norm_bins(u, min_b):
        m = u[0]
        for j in range(1, _BINS):
            m = jnp.maximum(m, u[j])
        e = [jnp.exp(u[j] - m) for j in range(_BINS)]
        s = e[0]
        for j in range(1, _BINS):
            s = s + e[j]
        rinv = 1.0 / s
        c = jnp.float32(1.0 - min_b * _BINS)
        return [jnp.float32(min_b) + c * e[j] * rinv for j in range(_BINS)]

    w = norm_bins(uw, _MIN_W)
    h = norm_bins(uh, _MIN_H)

    def knots(v):
        cw = [jnp.full((_L,), -_TAIL, jnp.float32)]
        s = v[0]
        for j in range(1, _BINS):
            cw.append(jnp.float32(2.0 * _TAIL) * s - _TAIL)
            s = s + v[j]
        cw.append(jnp.full((_L,), _TAIL, jnp.float32))
        return cw

    cw = knots(w)
    ch = knots(h)
    W = [cw[k + 1] - cw[k] for k in range(_BINS)]
    H = [ch[k + 1] - ch[k] for k in range(_BINS)]
    one = jnp.full((_L,), 1.0, jnp.float32)
    d = [one] + [jnp.float32(_MIN_D) + _softplus(u) for u in ud] + [one]

    yc = jnp.clip(y, -_TAIL, _TAIL)
    q_cw, q_W, q_ch, q_H, q_d, q_d1 = cw[0], W[0], ch[0], H[0], d[0], d[1]
    for k in range(1, _BINS):
        ge = yc >= cw[k]
        q_cw = jnp.where(ge, cw[k], q_cw)
        q_W = jnp.where(ge, W[k], q_W)
        q_ch = jnp.where(ge, ch[k], q_ch)
        q_H = jnp.where(ge, H[k], q_H)
        q_d = jnp.where(ge, d[k], q_d)
        q_d1 = jnp.where(ge, d[k + 1], q_d1)

    delta = q_H / q_W
    th = (yc - q_cw) / q_W
    th_om = th * (1.0 - th)
    th2 = th * th
    num = q_H * (delta * th2 + q_d * th_om)
    den = delta + (q_d + q_d1 - 2.0 * delta) * th_om
    out_in = q_ch + num / den
    omth = 1.0 - th
    dnum = delta * delta * (q_d1 * th2 + 2.0 * delta * th_om + q_d * omth * omth)
    ld_in = _log_f32(dnum / (den * den))

    inside = (y >= -_TAIL) & (y <= _TAIL)
    tu_v[pl.ds(g * _L, _L)] = jnp.where(inside, out_in, y)
    ld_v[pl.ds(g * _L, _L)] = jnp.where(inside, ld_in, jnp.float32(0.0))


def _sc_body(theta_hbm, y_hbm, tu_hbm, ld_hbm, theta_v, y_v, tu_v, ld_v):
    wid = lax.axis_index("s") * _NC + lax.axis_index("c")
    base = wid * _ROWS_PER_W

    def chunk_body(ci, carry):
        cbase = base + ci * _CHUNK
        pltpu.sync_copy(theta_hbm.at[pl.ds(cbase * _P, _CHUNK * _P)], theta_v)
        pltpu.sync_copy(y_hbm.at[pl.ds(cbase, _CHUNK)], y_v)

        def group_body(g, c):
            _rqs_group(theta_v, y_v, tu_v, ld_v, g)
            return c

        lax.fori_loop(0, _GROUPS, group_body, 0)
        pltpu.sync_copy(tu_v, tu_hbm.at[pl.ds(cbase, _CHUNK)])
        pltpu.sync_copy(ld_v, ld_hbm.at[pl.ds(cbase, _CHUNK)])
        return carry

    lax.fori_loop(0, _NCHUNK, chunk_body, 0)


_rqs_sc = functools.partial(
    pl.kernel,
    out_type=(
        jax.ShapeDtypeStruct((_N,), jnp.float32),
        jax.ShapeDtypeStruct((_N,), jnp.float32),
    ),
    mesh=plsc.VectorSubcoreMesh(core_axis_name="c", subcore_axis_name="s"),
    compiler_params=pltpu.CompilerParams(needs_layout_passes=False),
    scratch_types=[
        pltpu.VMEM((_CHUNK * _P,), jnp.float32),
        pltpu.VMEM((_CHUNK,), jnp.float32),
        pltpu.VMEM((_CHUNK,), jnp.float32),
        pltpu.VMEM((_CHUNK,), jnp.float32),
    ],
)(_sc_body)


_TCB = 32768
_TCS = 1024


def _softplus_tc(x):
    return jnp.maximum(x, 0.0) + jnp.log1p(jnp.exp(-jnp.abs(x)))


def _tc_body(theta_ref, y_ref, tu_ref, ld_ref):
    def sub_tile(i, carry):
        _tc_subtile(theta_ref, y_ref, tu_ref, ld_ref, i)
        return carry

    lax.fori_loop(0, _TCB // _TCS, sub_tile, 0, unroll=16)


def _tc_subtile(theta_ref, y_ref, tu_ref, ld_ref, i):
    t = theta_ref[pl.ds(i * _TCS, _TCS), :]
    f32 = jnp.float32
    tt = t.T
    uw = tt[0:8]
    uh = tt[8:16]
    udm = tt[17:24]
    y = y_ref[pl.ds(i * _TCS, _TCS)].reshape(1, _TCS)

    def norm_bins(u, scale, minb):
        e = jnp.exp(u)
        s = jnp.sum(e, axis=0, keepdims=True)
        rinv = f32((1.0 - minb * _BINS) * scale) / s
        return f32(minb * scale) + e * rinv

    w6 = norm_bins(uw, 2.0 * _TAIL, _MIN_W)
    h6 = norm_bins(uh, 2.0 * _TAIL, _MIN_H)

    i7 = lax.broadcasted_iota(jnp.int32, (7, _BINS), 0)
    j7 = lax.broadcasted_iota(jnp.int32, (7, _BINS), 1)
    ltri = (j7 <= i7).astype(f32)
    cs = lax.dot_general(ltri, w6, (((1,), (0,)), ((), ())),
                         preferred_element_type=f32,
                         precision=lax.Precision.HIGHEST)

    yc = jnp.clip(y, -_TAIL, _TAIL)
    z = yc + f32(_TAIL)
    ge = jnp.where(z >= cs, f32(1.0), f32(0.0))

    dm = f32(_MIN_D) + _softplus_tc(udm)
    one = jnp.ones((1, udm.shape[1]), f32)
    dmsh = jnp.concatenate([one, dm[0:6]], axis=0)
    dmsh1 = jnp.concatenate([dm[1:7], one], axis=0)

    def acc(diff):
        return jnp.sum(ge * diff, axis=0, keepdims=True)

    q_cw = acc(w6[0:7]) - f32(_TAIL)
    q_ch = acc(h6[0:7]) - f32(_TAIL)
    q_w = w6[0:1] + acc(w6[1:8] - w6[0:7])
    q_h = h6[0:1] + acc(h6[1:8] - h6[0:7])
    q_d = f32(1.0) + acc(dm - dmsh)
    q_d1 = dm[0:1] + acc(dmsh1 - dm)

    winv = f32(1.0) / q_w
    delta = q_h * winv
    th = (yc - q_cw) * winv
    th_om = th * (1.0 - th)
    th2 = th * th
    num = q_h * (delta * th2 + q_d * th_om)
    den = delta + (q_d + q_d1 - 2.0 * delta) * th_om
    deninv = f32(1.0) / den
    out_in = q_ch + num * deninv
    omth = 1.0 - th
    dnum = delta * delta * (q_d1 * th2 + 2.0 * delta * th_om + q_d * omth * omth)
    ld_in = jnp.log(dnum * deninv * deninv)

    inside = (y >= -_TAIL) & (y <= _TAIL)
    tu_ref[pl.ds(i * _TCS, _TCS)] = jnp.where(inside, out_in, y).reshape(_TCS)
    ld_ref[pl.ds(i * _TCS, _TCS)] = jnp.where(inside, ld_in, f32(0.0)).reshape(_TCS)


def _rqs_tc(theta, y):
    n = theta.shape[0]
    nb = n // _TCB
    tu, ld = pl.pallas_call(
        _tc_body,
        grid=(nb,),
        in_specs=[
            pl.BlockSpec((_TCB, _P), lambda i: (i, 0)),
            pl.BlockSpec((_TCB,), lambda i: (i,)),
        ],
        out_specs=[
            pl.BlockSpec((_TCB,), lambda i: (i,)),
            pl.BlockSpec((_TCB,), lambda i: (i,)),
        ],
        out_shape=[
            jax.ShapeDtypeStruct((n,), jnp.float32),
            jax.ShapeDtypeStruct((n,), jnp.float32),
        ],
    )(theta, y)
    return tu, ld


def kernel(theta, y):
    tu, ld = _rqs_tc(theta, y)
    return tu.reshape(-1, 1), ld
